# Initial kernel scaffold; baseline (speedup 1.0000x reference)
import jax
import jax.numpy as jnp
from jax.experimental import pallas as pl


def kernel(y, e, mu_causal, cov_causal, mu_spurious, cov_spurious):
    raise NotImplementedError("write your pallas kernel here")


# speedup vs baseline: 3.1955x; 3.1955x over previous
import functools

import numpy as np
import jax
import jax.numpy as jnp
from jax import lax
from jax.experimental import pallas as pl
from jax.experimental.pallas import tpu as pltpu
from jax.experimental.pallas import tpu_sc as plsc

_Z = 32
_NT = _Z * (_Z + 1) // 2
_B = 4096
_NE = 64
_NC = 1000


def _build_scatter_matrix():
    s = np.zeros((_NT, _Z * _Z), dtype=np.float32)
    t = 0
    for i in range(_Z):
        for j in range(i + 1):
            s[t, i * _Z + j] = 1.0
            t += 1
    return s


_SCATTER_NP = _build_scatter_matrix()


def _sc_gather(y_flat, e_flat, mu_sp_flat, cov_sp_flat):
    info = plsc.get_sparse_core_info()
    num_cores, num_subcores = info.num_cores, info.num_subcores
    nw = num_cores * num_subcores
    bpw = _B // nw
    lanes = info.num_lanes

    mesh = plsc.VectorSubcoreMesh(core_axis_name="c", subcore_axis_name="s")

    @functools.partial(
        pl.kernel,
        out_type=(
            jax.ShapeDtypeStruct((_B, _Z), jnp.float32),
            jax.ShapeDtypeStruct((_B, _NT), jnp.float32),
        ),
        mesh=mesh,
        scratch_types=[
            pltpu.VMEM((bpw,), jnp.int32),
            pltpu.VMEM((bpw,), jnp.int32),
            pltpu.VMEM((bpw,), jnp.int32),
            pltpu.VMEM((bpw, _Z), jnp.float32),
            pltpu.VMEM((bpw, _NT), jnp.float32),
            pltpu.SemaphoreType.DMA,
            pltpu.SemaphoreType.DMA,
        ],
        compiler_params=pltpu.CompilerParams(use_tc_tiling_on_sc=False),
    )
    def gather_kernel(y_hbm, e_hbm, mu_hbm, cov_hbm, mu_out, cov_out,
                      y_v, e_v, idx_v, mu_rows, cov_rows, sem_mu, sem_cov):
        wid = lax.axis_index("s") * num_cores + lax.axis_index("c")
        base = wid * bpw
        pltpu.sync_copy(y_hbm.at[pl.ds(base, bpw)], y_v)
        pltpu.sync_copy(e_hbm.at[pl.ds(base, bpw)], e_v)
        for i in range(bpw // lanes):
            sl = pl.ds(i * lanes, lanes)
            idx_v[sl] = y_v[sl] * _NE + e_v[sl]
        cp_mu = pltpu.async_copy(mu_hbm.at[idx_v], mu_rows, sem_mu)
        cp_cov = pltpu.async_copy(cov_hbm.at[idx_v], cov_rows, sem_cov)
        cp_mu.wait()
        cp_cov.wait()
        pltpu.sync_copy(mu_rows, mu_out.at[pl.ds(base, bpw)])
        pltpu.sync_copy(cov_rows, cov_out.at[pl.ds(base, bpw)])

    return gather_kernel(y_flat, e_flat, mu_sp_flat, cov_sp_flat)


def _tc_assemble(e_col, mu_causal, cov_causal, mu_s_rows, cov_s_rows,
                 scatter, interpret=False):
    bb = 256
    grid = _B // bb

    def body(e_ref, muc_ref, covc_ref, mus_ref, covs_ref, s_ref,
             mu_out_ref, cov_out_ref, cc_tab_ref):
        @pl.when(pl.program_id(0) == 0)
        def _():
            lc = jnp.reshape(
                jnp.dot(covc_ref[...], s_ref[...],
                        preferred_element_type=jnp.float32),
                (_NE, _Z, _Z))
            cc = lax.dot_general(lc, lc, (((2,), (2,)), ((0,), (0,))),
                                 preferred_element_type=jnp.float32)
            cc_tab_ref[...] = jnp.reshape(cc, (_NE, _Z * _Z))

        onehot = (e_ref[...] == lax.broadcasted_iota(
            jnp.int32, (bb, _NE), 1)).astype(jnp.float32)
        mu_c = jnp.dot(onehot, muc_ref[...], preferred_element_type=jnp.float32)
        mu_out_ref[...] = jnp.concatenate([mu_c, mus_ref[...]], axis=1)

        cov_c = jnp.reshape(
            jnp.dot(onehot, cc_tab_ref[...], preferred_element_type=jnp.float32),
            (bb, _Z, _Z))
        ls = jnp.reshape(
            jnp.dot(covs_ref[...], s_ref[...], preferred_element_type=jnp.float32),
            (bb, _Z, _Z))
        cov_s = lax.dot_general(ls, ls, (((2,), (2,)), ((0,), (0,))),
                                preferred_element_type=jnp.float32)
        zero = jnp.zeros((bb, _Z, _Z), jnp.float32)
        cov_out_ref[...] = jnp.concatenate([
            jnp.concatenate([cov_c, zero], axis=2),
            jnp.concatenate([zero, cov_s], axis=2),
        ], axis=1)

    return pl.pallas_call(
        body,
        grid=(grid,),
        in_specs=[
            pl.BlockSpec((bb, 1), lambda i: (i, 0)),
            pl.BlockSpec((_NE, _Z), lambda i: (0, 0)),
            pl.BlockSpec((_NE, _NT), lambda i: (0, 0)),
            pl.BlockSpec((bb, _Z), lambda i: (i, 0)),
            pl.BlockSpec((bb, _NT), lambda i: (i, 0)),
            pl.BlockSpec((_NT, _Z * _Z), lambda i: (0, 0)),
        ],
        out_specs=[
            pl.BlockSpec((bb, 2 * _Z), lambda i: (i, 0)),
            pl.BlockSpec((bb, 2 * _Z, 2 * _Z), lambda i: (i, 0, 0)),
        ],
        out_shape=[
            jax.ShapeDtypeStruct((_B, 2 * _Z), jnp.float32),
            jax.ShapeDtypeStruct((_B, 2 * _Z, 2 * _Z), jnp.float32),
        ],
        scratch_shapes=[pltpu.VMEM((_NE, _Z * _Z), jnp.float32)],
        interpret=interpret,
    )(e_col, mu_causal, cov_causal, mu_s_rows, cov_s_rows, scatter)


def kernel(y, e, mu_causal, cov_causal, mu_spurious, cov_spurious):
    y_flat = y.reshape(_B).astype(jnp.int32)
    e_flat = e.reshape(_B).astype(jnp.int32)
    mu_sp_flat = mu_spurious.reshape(_NC * _NE, _Z)
    cov_sp_flat = cov_spurious.reshape(_NC * _NE, _NT)
    scatter = jnp.asarray(_SCATTER_NP)

    mu_s_rows, cov_s_rows = _sc_gather(y_flat, e_flat, mu_sp_flat, cov_sp_flat)
    mu, cov = _tc_assemble(e.astype(jnp.int32), mu_causal, cov_causal,
                           mu_s_rows, cov_s_rows, scatter)
    return (mu, cov)


# speedup vs baseline: 3.3609x; 1.0518x over previous
import functools

import numpy as np
import jax
import jax.numpy as jnp
from jax import lax
from jax.experimental import pallas as pl
from jax.experimental.pallas import tpu as pltpu
from jax.experimental.pallas import tpu_sc as plsc

_Z = 32
_NT = _Z * (_Z + 1) // 2
_B = 4096
_NE = 64
_NC = 1000


def _build_scatter_matrix():
    s = np.zeros((_NT, _Z * _Z), dtype=np.float32)
    t = 0
    for i in range(_Z):
        for j in range(i + 1):
            s[t, i * _Z + j] = 1.0
            t += 1
    return s


_SCATTER_NP = _build_scatter_matrix()


_ROW = 768


def _sc_gather(y_flat, e_flat, table):
    info = plsc.get_sparse_core_info()
    num_cores, num_subcores = info.num_cores, info.num_subcores
    nw = num_cores * num_subcores
    bpw = _B // nw
    lanes = info.num_lanes

    mesh = plsc.VectorSubcoreMesh(core_axis_name="c", subcore_axis_name="s")

    @functools.partial(
        pl.kernel,
        out_type=jax.ShapeDtypeStruct((_B, _ROW), jnp.float32),
        mesh=mesh,
        scratch_types=[
            pltpu.VMEM((bpw,), jnp.int32),
            pltpu.VMEM((bpw,), jnp.int32),
            pltpu.VMEM((bpw,), jnp.int32),
            pltpu.VMEM((bpw, _ROW), jnp.float32),
            pltpu.SemaphoreType.DMA,
        ],
    )
    def gather_kernel(y_hbm, e_hbm, tab_hbm, rows_out,
                      y_v, e_v, idx_v, rows_v, sem):
        wid = lax.axis_index("s") * num_cores + lax.axis_index("c")
        base = wid * bpw
        pltpu.sync_copy(y_hbm.at[pl.ds(base, bpw)], y_v)
        pltpu.sync_copy(e_hbm.at[pl.ds(base, bpw)], e_v)
        for i in range(bpw // lanes):
            sl = pl.ds(i * lanes, lanes)
            idx_v[sl] = y_v[sl] * _NE + e_v[sl]
        pltpu.async_copy(tab_hbm.at[idx_v], rows_v, sem).wait()
        pltpu.sync_copy(rows_v, rows_out.at[pl.ds(base, bpw)])

    return gather_kernel(y_flat, e_flat, table)


def _tc_assemble(e_col, mu_causal, cov_causal, sp_rows, scatter,
                 interpret=False):
    bb = 256
    grid = _B // bb

    def body(e_ref, muc_ref, covc_ref, rows_ref, s_ref,
             mu_out_ref, cov_out_ref, cc_tab_ref):
        @pl.when(pl.program_id(0) == 0)
        def _():
            lc = jnp.reshape(
                jnp.dot(covc_ref[...], s_ref[...],
                        preferred_element_type=jnp.float32),
                (_NE, _Z, _Z))
            cc = lax.dot_general(lc, lc, (((2,), (2,)), ((0,), (0,))),
                                 preferred_element_type=jnp.float32)
            cc_tab_ref[...] = jnp.reshape(cc, (_NE, _Z * _Z))

        onehot = (e_ref[...] == lax.broadcasted_iota(
            jnp.int32, (bb, _NE), 1)).astype(jnp.float32)
        mu_c = jnp.dot(onehot, muc_ref[...], preferred_element_type=jnp.float32)
        mu_s = rows_ref[:, 640:640 + _Z]
        mu_out_ref[...] = jnp.concatenate([mu_c, mu_s], axis=1)

        cov_c = jnp.reshape(
            jnp.dot(onehot, cc_tab_ref[...], preferred_element_type=jnp.float32),
            (bb, _Z, _Z))
        ls = jnp.reshape(
            jnp.dot(rows_ref[:, 0:_NT], s_ref[...],
                    preferred_element_type=jnp.float32),
            (bb, _Z, _Z))
        cov_s = lax.dot_general(ls, ls, (((2,), (2,)), ((0,), (0,))),
                                preferred_element_type=jnp.float32)
        zero = jnp.zeros((bb, _Z, _Z), jnp.float32)
        cov_out_ref[...] = jnp.concatenate([
            jnp.concatenate([cov_c, zero], axis=2),
            jnp.concatenate([zero, cov_s], axis=2),
        ], axis=1)

    return pl.pallas_call(
        body,
        grid=(grid,),
        in_specs=[
            pl.BlockSpec((bb, 1), lambda i: (i, 0)),
            pl.BlockSpec((_NE, _Z), lambda i: (0, 0)),
            pl.BlockSpec((_NE, _NT), lambda i: (0, 0)),
            pl.BlockSpec((bb, _ROW), lambda i: (i, 0)),
            pl.BlockSpec((_NT, _Z * _Z), lambda i: (0, 0)),
        ],
        out_specs=[
            pl.BlockSpec((bb, 2 * _Z), lambda i: (i, 0)),
            pl.BlockSpec((bb, 2 * _Z, 2 * _Z), lambda i: (i, 0, 0)),
        ],
        out_shape=[
            jax.ShapeDtypeStruct((_B, 2 * _Z), jnp.float32),
            jax.ShapeDtypeStruct((_B, 2 * _Z, 2 * _Z), jnp.float32),
        ],
        scratch_shapes=[pltpu.VMEM((_NE, _Z * _Z), jnp.float32)],
        interpret=interpret,
    )(e_col, mu_causal, cov_causal, sp_rows, scatter)


def kernel(y, e, mu_causal, cov_causal, mu_spurious, cov_spurious):
    y_flat = y.reshape(_B).astype(jnp.int32)
    e_flat = e.reshape(_B).astype(jnp.int32)
    mu_sp_flat = mu_spurious.reshape(_NC * _NE, _Z)
    cov_sp_flat = cov_spurious.reshape(_NC * _NE, _NT)
    scatter = jnp.asarray(_SCATTER_NP)

    table = jnp.concatenate([
        cov_sp_flat,
        jnp.zeros((_NC * _NE, 640 - _NT), jnp.float32),
        mu_sp_flat,
        jnp.zeros((_NC * _NE, _ROW - 640 - _Z), jnp.float32),
    ], axis=1)

    sp_rows = _sc_gather(y_flat, e_flat, table)
    mu, cov = _tc_assemble(e.astype(jnp.int32), mu_causal, cov_causal,
                           sp_rows, scatter)
    return (mu, cov)
